# Initial kernel scaffold; baseline (speedup 1.0000x reference)
import jax
import jax.numpy as jnp
from jax.experimental import pallas as pl


def kernel(embedded_nodes, encoded_nodes, Wc1, bc1, Wc2, bc2, Wn1, Bn1, Wn2, Bn2, We1, Be1, We2, Be2, bn_super_w, bn_super_b, bn_bip_w, bn_bip_b, centroids):
    raise NotImplementedError("write your pallas kernel here")


# speedup vs baseline: 4.1224x; 4.1224x over previous
import functools

import jax
import jax.numpy as jnp
from jax import lax
from jax.experimental import pallas as pl
from jax.experimental.pallas import tpu as pltpu
from jax.experimental.pallas import tpu_sc as plsc

N = 50000
LATENT = 128
HIDDEN = 256
EMB = 24
C = 1000
K_SUPER = 10
K_BIP = 3
RSQ = 1.0

NW = 32

MEANS_CHUNKS = 13
MEANS_ROWS_W = MEANS_CHUNKS * 128
N_PAD1 = NW * MEANS_ROWS_W

SUP_CHUNKS = 37
SUP_ROWS_W = SUP_CHUNKS * 128
E_PAD = NW * SUP_ROWS_W

BLK = 400
NBLK = N // BLK


def _k1_body(x_ref, e_ref, noise_ref, wc1_ref, bc1_ref, wc2_ref, bc2_ref,
             wn1_ref, bn1_ref, wn2_ref, bn2_ref, centt_ref,
             emb_ref, cl_ref, msg_ref):
    x = x_ref[...]
    h = jnp.maximum(x @ wc1_ref[...] + bc1_ref[...], 0.0)
    o = h @ wc2_ref[...] + bc2_ref[...]
    nrm = jnp.sqrt(jnp.sum(o * o, axis=1, keepdims=True))
    emb = o / jnp.maximum(nrm, 1e-12)
    emb_ref[...] = emb
    logits = (emb * noise_ref[...]) @ centt_ref[...]
    mx = jnp.max(logits, axis=1, keepdims=True)
    iota = lax.broadcasted_iota(jnp.int32, logits.shape, 1)
    cl = jnp.min(jnp.where(logits == mx, iota, C), axis=1)
    cl_ref[...] = cl[:, None]
    e = e_ref[...]
    h2 = jnp.maximum(e @ wn1_ref[...] + bn1_ref[...], 0.0)
    msg_ref[...] = jnp.maximum(h2 @ wn2_ref[...] + bn2_ref[...], 0.0)


def _run_k1(embedded, encoded, noise, Wc1, bc1, Wc2, bc2, Wn1, Bn1, Wn2, Bn2,
            centT):
    full = lambda b: (0, 0)
    return pl.pallas_call(
        _k1_body,
        grid=(NBLK,),
        in_specs=[
            pl.BlockSpec((BLK, LATENT), lambda b: (b, 0)),
            pl.BlockSpec((BLK, LATENT), lambda b: (b, 0)),
            pl.BlockSpec((BLK, 1), lambda b: (b, 0)),
            pl.BlockSpec((LATENT, HIDDEN), full),
            pl.BlockSpec((1, HIDDEN), full),
            pl.BlockSpec((HIDDEN, EMB), full),
            pl.BlockSpec((1, EMB), full),
            pl.BlockSpec((LATENT, HIDDEN), full),
            pl.BlockSpec((1, HIDDEN), full),
            pl.BlockSpec((HIDDEN, LATENT), full),
            pl.BlockSpec((1, LATENT), full),
            pl.BlockSpec((EMB, C), full),
        ],
        out_specs=[
            pl.BlockSpec((BLK, EMB), lambda b: (b, 0)),
            pl.BlockSpec((BLK, 1), lambda b: (b, 0)),
            pl.BlockSpec((BLK, LATENT), lambda b: (b, 0)),
        ],
        out_shape=[
            jax.ShapeDtypeStruct((N, EMB), jnp.float32),
            jax.ShapeDtypeStruct((N, 1), jnp.int32),
            jax.ShapeDtypeStruct((N, LATENT), jnp.float32),
        ],
    )(embedded, encoded, noise, Wc1, bc1, Wc2, bc2, Wn1, Bn1, Wn2, Bn2, centT)


def _make_sc_segsum(n_chunks, rows_w, d):
    mesh = plsc.VectorSubcoreMesh(core_axis_name="c", subcore_axis_name="s")

    @functools.partial(
        pl.kernel,
        out_type=jax.ShapeDtypeStruct((2, C, d), jnp.float32),
        mesh=mesh,
        scratch_types=[
            pltpu.VMEM((n_chunks, 128), jnp.int32),
            pltpu.VMEM((128, d), jnp.float32),
            pltpu.VMEM_SHARED((C, d), jnp.float32),
        ],
    )
    def segsum(data_hbm, idx_hbm, zero_hbm, out_hbm, idx_v, buf_v, acc_sh):
        cid = lax.axis_index("c")
        sid = lax.axis_index("s")
        wid = sid * 2 + cid

        @pl.when(sid == 0)
        def _init():
            pltpu.sync_copy(zero_hbm, acc_sh)

        plsc.subcore_barrier()
        pltpu.sync_copy(idx_hbm.at[wid], idx_v)

        def body(j, carry):
            base = wid * rows_w + j * 128
            pltpu.sync_copy(data_hbm.at[pl.ds(base, 128)], buf_v)
            pltpu.sync_copy(buf_v, acc_sh.at[idx_v.at[j]], add=True)
            return carry

        lax.fori_loop(0, n_chunks, body, 0)
        plsc.subcore_barrier()

        @pl.when(sid == 0)
        def _flush():
            pltpu.sync_copy(acc_sh, out_hbm.at[cid])

    return segsum


@functools.lru_cache(maxsize=None)
def _sc_segsum_cached(n_chunks, rows_w, d):
    return _make_sc_segsum(n_chunks, rows_w, d)


def _sc_segsum_means(data, idx, zero):
    return _sc_segsum_cached(MEANS_CHUNKS, MEANS_ROWS_W, 128)(data, idx, zero)


def _sc_segsum_super(data, idx, zero):
    return _sc_segsum_cached(SUP_CHUNKS, SUP_ROWS_W, LATENT)(data, idx, zero)


def _k3a_body(p_ref, means_ref):
    p = p_ref[0] + p_ref[1]
    s = p[:, :EMB]
    cnt = p[:, EMB:EMB + 1]
    m = s / jnp.maximum(cnt, 1.0)
    nrm = jnp.sqrt(jnp.sum(m * m, axis=1, keepdims=True))
    means_ref[...] = m / jnp.maximum(nrm, 1e-12)


def _run_k3a(partials):
    return pl.pallas_call(
        _k3a_body,
        out_shape=jax.ShapeDtypeStruct((C, EMB), jnp.float32),
    )(partials)


def _k3b_body(means_ref, meanst_ref, g_ref, b_ref, idx_ref, w_ref):
    means = means_ref[...]
    g = means @ meanst_ref[...]
    ss = jnp.sum(means * means, axis=1, keepdims=True)
    sst = jnp.sum(meanst_ref[...] * meanst_ref[...], axis=0, keepdims=True)
    d2 = ss - 2.0 * g + sst
    cur = -d2
    iota = lax.broadcasted_iota(jnp.int32, (C, C), 1)
    idxs, liks, vals = [], [], []
    for _ in range(K_SUPER):
        mx = jnp.max(cur, axis=1, keepdims=True)
        sel = jnp.min(jnp.where(cur == mx, iota, C), axis=1)
        onehot = iota == sel[:, None]
        idxs.append(sel[:, None])
        liks.append(jnp.sum(jnp.where(onehot, g, 0.0), axis=1, keepdims=True))
        vals.append((-mx <= RSQ).astype(jnp.float32))
        cur = jnp.where(onehot, -jnp.inf, cur)
    lik = jnp.concatenate(liks, axis=1)
    val = jnp.concatenate(vals, axis=1)
    m = jnp.mean(lik)
    v = jnp.mean((lik - m) ** 2)
    z = (lik - m) / jnp.sqrt(v + 1e-5) * g_ref[0, 0] + b_ref[0, 0]
    w_ref[...] = jax.nn.sigmoid(z) * val
    idx_ref[...] = jnp.concatenate(idxs, axis=1)


def _run_k3b(means, meansT, bnw, bnb):
    return pl.pallas_call(
        _k3b_body,
        out_shape=[
            jax.ShapeDtypeStruct((C, K_SUPER), jnp.int32),
            jax.ShapeDtypeStruct((C, K_SUPER), jnp.float32),
        ],
    )(means, meansT, bnw, bnb)


def _k4_body(emb_ref, meanst_ref, idx_ref, lik_ref, val_ref, part_ref):
    e = emb_ref[...]
    logits = e @ meanst_ref[...]
    ss = jnp.sum(e * e, axis=1, keepdims=True)
    sst = jnp.sum(meanst_ref[...] * meanst_ref[...], axis=0, keepdims=True)
    d2 = ss - 2.0 * logits + sst
    cur = -d2
    iota = lax.broadcasted_iota(jnp.int32, (BLK, C), 1)
    idxs, liks, vals = [], [], []
    for _ in range(K_BIP):
        mx = jnp.max(cur, axis=1, keepdims=True)
        sel = jnp.min(jnp.where(cur == mx, iota, C), axis=1)
        onehot = iota == sel[:, None]
        idxs.append(sel[:, None])
        liks.append(jnp.sum(jnp.where(onehot, logits, 0.0), axis=1,
                            keepdims=True))
        vals.append((-mx <= RSQ).astype(jnp.float32))
        cur = jnp.where(onehot, -jnp.inf, cur)
    lik = jnp.concatenate(liks, axis=1)
    idx_ref[...] = jnp.concatenate(idxs, axis=1)
    lik_ref[...] = lik
    val_ref[...] = jnp.concatenate(vals, axis=1)
    s = jnp.sum(lik)
    s2 = jnp.sum(lik * lik)
    part_ref[...] = jnp.concatenate(
        [jnp.full((1, 1, 1), s), jnp.full((1, 1, 1), s2),
         jnp.zeros((1, 1, 6), jnp.float32)], axis=2)


def _run_k4(emb, meansT):
    return pl.pallas_call(
        _k4_body,
        grid=(NBLK,),
        in_specs=[
            pl.BlockSpec((BLK, EMB), lambda b: (b, 0)),
            pl.BlockSpec((EMB, C), lambda b: (0, 0)),
        ],
        out_specs=[
            pl.BlockSpec((BLK, K_BIP), lambda b: (b, 0)),
            pl.BlockSpec((BLK, K_BIP), lambda b: (b, 0)),
            pl.BlockSpec((BLK, K_BIP), lambda b: (b, 0)),
            pl.BlockSpec((1, 1, 8), lambda b: (b, 0, 0)),
        ],
        out_shape=[
            jax.ShapeDtypeStruct((N, K_BIP), jnp.int32),
            jax.ShapeDtypeStruct((N, K_BIP), jnp.float32),
            jax.ShapeDtypeStruct((N, K_BIP), jnp.float32),
            jax.ShapeDtypeStruct((NBLK, 1, 8), jnp.float32),
        ],
    )(emb, meansT)


def _k5_body(lik_ref, val_ref, msg_ref, part_ref, g_ref, b_ref,
             wn_ref, scaled_ref):
    p = part_ref[...]
    s = jnp.sum(p[:, :, 0])
    s2 = jnp.sum(p[:, :, 1])
    cnt = float(N * K_BIP)
    m = s / cnt
    v = s2 / cnt - m * m
    lik = lik_ref[...]
    z = (lik - m) / jnp.sqrt(v + 1e-5) * g_ref[0, 0] + b_ref[0, 0]
    w = jnp.exp(z) * val_ref[...]
    den = jnp.sum(w, axis=1, keepdims=True)
    wn = w / (1e-12 + den)
    wn_ref[...] = wn
    msg = msg_ref[...]
    scaled_ref[...] = jnp.concatenate(
        [msg * wn[:, k:k + 1] for k in range(K_BIP)], axis=1)


def _run_k5(lik, val, msg, part, bnw, bnb):
    return pl.pallas_call(
        _k5_body,
        grid=(NBLK,),
        in_specs=[
            pl.BlockSpec((BLK, K_BIP), lambda b: (b, 0)),
            pl.BlockSpec((BLK, K_BIP), lambda b: (b, 0)),
            pl.BlockSpec((BLK, LATENT), lambda b: (b, 0)),
            pl.BlockSpec((NBLK, 1, 8), lambda b: (0, 0, 0)),
            pl.BlockSpec((1, 1), lambda b: (0, 0)),
            pl.BlockSpec((1, 1), lambda b: (0, 0)),
        ],
        out_specs=[
            pl.BlockSpec((BLK, K_BIP), lambda b: (b, 0)),
            pl.BlockSpec((BLK, K_BIP * LATENT), lambda b: (b, 0)),
        ],
        out_shape=[
            jax.ShapeDtypeStruct((N, K_BIP), jnp.float32),
            jax.ShapeDtypeStruct((N, K_BIP * LATENT), jnp.float32),
        ],
    )(lik, val, msg, part, bnw, bnb)


SE_BLK_C = 200
SE_BLK_E = SE_BLK_C * K_SUPER


def _k7_body(pb_ref, pf_ref, idx_ref, we1_ref, be1_ref, we2_ref, be2_ref,
             sup_ref, se1_ref, se2_ref):
    sn_full = pf_ref[0] + pf_ref[1]
    snblk = pb_ref[0] + pb_ref[1]
    sup_ref[...] = snblk
    idxv = idx_ref[0]
    iota_c = lax.broadcasted_iota(jnp.int32, (SE_BLK_E, C), 1)
    gsel = (iota_c == idxv).astype(jnp.float32)
    gat = gsel @ sn_full
    iota_e = lax.broadcasted_iota(jnp.int32, (SE_BLK_E, SE_BLK_C), 0)
    iota_i = lax.broadcasted_iota(jnp.int32, (SE_BLK_E, SE_BLK_C), 1)
    rsel = ((iota_e // K_SUPER) == iota_i).astype(jnp.float32)
    rep = rsel @ snblk
    we1 = we1_ref[...]
    w1a = we1[:LATENT]
    w1b = we1[LATENT:]
    be1 = be1_ref[...]
    h1 = jnp.maximum(rep @ w1a + gat @ w1b + be1, 0.0)
    se1_ref[...] = jnp.maximum(h1 @ we2_ref[...] + be2_ref[...], 0.0)
    h2 = jnp.maximum(gat @ w1a + rep @ w1b + be1, 0.0)
    se2_ref[...] = jnp.maximum(h2 @ we2_ref[...] + be2_ref[...], 0.0)


def _run_k7(partials, idx3, We1, Be1, We2, Be2):
    nb = C // SE_BLK_C
    ne = C * K_SUPER
    return pl.pallas_call(
        _k7_body,
        grid=(nb,),
        in_specs=[
            pl.BlockSpec((2, SE_BLK_C, LATENT), lambda b: (0, b, 0)),
            pl.BlockSpec((2, C, LATENT), lambda b: (0, 0, 0)),
            pl.BlockSpec((1, SE_BLK_E, 1), lambda b: (b, 0, 0)),
            pl.BlockSpec((2 * LATENT, HIDDEN), lambda b: (0, 0)),
            pl.BlockSpec((1, HIDDEN), lambda b: (0, 0)),
            pl.BlockSpec((HIDDEN, LATENT), lambda b: (0, 0)),
            pl.BlockSpec((1, LATENT), lambda b: (0, 0)),
        ],
        out_specs=[
            pl.BlockSpec((SE_BLK_C, LATENT), lambda b: (b, 0)),
            pl.BlockSpec((SE_BLK_E, LATENT), lambda b: (b, 0)),
            pl.BlockSpec((SE_BLK_E, LATENT), lambda b: (b, 0)),
        ],
        out_shape=[
            jax.ShapeDtypeStruct((C, LATENT), jnp.float32),
            jax.ShapeDtypeStruct((ne, LATENT), jnp.float32),
            jax.ShapeDtypeStruct((ne, LATENT), jnp.float32),
        ],
    )(partials, partials, idx3, We1, Be1, We2, Be2)


def kernel(embedded_nodes, encoded_nodes, Wc1, bc1, Wc2, bc2, Wn1, Bn1, Wn2,
           Bn2, We1, Be1, We2, Be2, bn_super_w, bn_super_b, bn_bip_w,
           bn_bip_b, centroids):
    f32 = jnp.float32
    noise = 1.0 + 0.005 * jax.random.normal(jax.random.key(42), (N,), f32)
    emb, cl, msg = _run_k1(
        embedded_nodes, encoded_nodes, noise[:, None],
        Wc1, bc1[None, :], Wc2, bc2[None, :],
        Wn1, Bn1[None, :], Wn2, Bn2[None, :], centroids.T)

    data1 = jnp.pad(
        jnp.concatenate([emb, jnp.ones((N, 1), f32)], axis=1),
        ((0, N_PAD1 - N), (0, 128 - EMB - 1)))
    idx1 = jnp.pad(cl[:, 0], (0, N_PAD1 - N)).reshape(NW, MEANS_CHUNKS, 128)
    part1 = _sc_segsum_means(data1, idx1, jnp.zeros((C, 128), f32))

    means = _run_k3a(part1)
    meansT = means.T
    idx_s, w_sup = _run_k3b(means, meansT,
                            bn_super_w[:, None], bn_super_b[:, None])

    idx_b, lik_b, val_b, part_b = _run_k4(emb, meansT)
    wn, scaled = _run_k5(lik_b, val_b, msg, part_b,
                         bn_bip_w[:, None], bn_bip_b[:, None])

    rows = scaled.reshape(N * K_BIP, LATENT)
    data2 = jnp.pad(rows, ((0, E_PAD - N * K_BIP), (0, 0)))
    idx2 = jnp.pad(idx_b.reshape(-1),
                   (0, E_PAD - N * K_BIP)).reshape(NW, SUP_CHUNKS, 128)
    part2 = _sc_segsum_super(data2, idx2, jnp.zeros((C, LATENT), f32))

    idx3 = idx_s.reshape(C // SE_BLK_C, SE_BLK_E, 1)
    supernodes, se1, se2 = _run_k7(part2, idx3, We1, Be1[None, :],
                                   We2, Be2[None, :])
    superedges = jnp.concatenate([se1, se2], axis=0)

    src_s = jnp.repeat(jnp.arange(C, dtype=jnp.int32), K_SUPER)
    dst_s = idx_s.reshape(-1)
    sg = jnp.stack([jnp.concatenate([src_s, dst_s]),
                    jnp.concatenate([dst_s, src_s])])
    wf = w_sup.reshape(-1)
    super_edge_weights = jnp.concatenate([wf, wf])[:, None]

    src_b = jnp.repeat(jnp.arange(N, dtype=jnp.int32), K_BIP)
    bg = jnp.stack([src_b, idx_b.reshape(-1)])
    bipartite_edge_weights = wn.reshape(-1)[:, None]

    return (emb, supernodes, superedges, bg, bipartite_edge_weights, sg,
            super_edge_weights)
